# baseline (speedup 1.0000x reference)
import jax
import jax.numpy as jnp
from jax.experimental import pallas as pl
from jax.experimental.pallas import tpu as pltpu

_L = 512


def _attn_kernel(cu_ref, embs_ref, w_ref, b_ref, diag_ref, out_ref):
    b = pl.program_id(0)
    t = embs_ref.shape[0]
    d = out_ref.shape[1]
    start = cu_ref[b]
    end = cu_ref[b + 1]
    sc = jnp.minimum(start, t - _L)

    x = embs_ref[pl.ds(sc, _L), :]
    qkv = jnp.dot(x, w_ref[...], preferred_element_type=jnp.float32) + b_ref[...]
    q = qkv[:, :d]
    k = qkv[:, d:2 * d]
    v = qkv[:, 2 * d:]

    col1 = sc + jax.lax.broadcasted_iota(jnp.int32, (1, _L), 1)
    cb = jnp.where((col1 >= start) & (col1 < end), 0.0, -1e30)

    s = jax.lax.dot_general(q, k, (((1,), (1,)), ((), ())),
                            preferred_element_type=jnp.float32)
    s = s + diag_ref[...] + cb
    m = jnp.max(s, axis=1, keepdims=True)
    p = jnp.exp(s - m)
    denom = jnp.sum(p, axis=1, keepdims=True)
    o = jnp.dot(p, v, preferred_element_type=jnp.float32) / denom

    row1 = sc + jax.lax.broadcasted_iota(jnp.int32, (_L, 1), 0)
    row_valid = (row1 >= start) & (row1 < end)
    cur = out_ref[pl.ds(sc, _L), :]
    out_ref[pl.ds(sc, _L), :] = jnp.where(row_valid, o, cur)


def kernel(embs_local_global, cu_seqlens, Wq, Wk, Wv, bq, bk, bv):
    t, d = embs_local_global.shape
    nseg = cu_seqlens.shape[0] - 1
    w = jnp.concatenate([Wq, Wk, Wv], axis=1)
    bias = jnp.concatenate([bq, bk, bv]).reshape(1, 3 * d)
    ii = jax.lax.broadcasted_iota(jnp.int32, (_L, _L), 0)
    jj = jax.lax.broadcasted_iota(jnp.int32, (_L, _L), 1)
    diag_bias = jnp.where(ii == jj, -1e30, 0.0).astype(jnp.float32)
    full = lambda shape: pl.BlockSpec(shape, lambda b: (0,) * len(shape))
    return pl.pallas_call(
        _attn_kernel,
        grid=(nseg,),
        in_specs=[
            pl.BlockSpec(memory_space=pltpu.SMEM),
            full((t, d)),
            full((d, 3 * d)),
            full((1, 3 * d)),
            full((_L, _L)),
        ],
        out_specs=full((t, d)),
        out_shape=jax.ShapeDtypeStruct((t, d), jnp.float32),
        compiler_params=pltpu.CompilerParams(
            dimension_semantics=("arbitrary",)),
    )(cu_seqlens, embs_local_global, w, bias, diag_bias)


# speedup vs baseline: 1.0646x; 1.0646x over previous
import jax
import jax.numpy as jnp
from jax.experimental import pallas as pl
from jax.experimental.pallas import tpu as pltpu

_L = 512


def _attn_kernel(cu_ref, embs_ref, w_ref, b_ref, out_ref):
    b = pl.program_id(0)
    t = embs_ref.shape[0]
    d = out_ref.shape[1]
    start = cu_ref[b]
    end = cu_ref[b + 1]
    sc = jnp.minimum(start, t - _L)

    x = embs_ref[pl.ds(sc, _L), :]
    qkv = jnp.dot(x, w_ref[...], preferred_element_type=jnp.float32) + b_ref[...]
    q = qkv[:, :d]
    k = qkv[:, d:2 * d]
    v = qkv[:, 2 * d:]

    col1 = sc + jax.lax.broadcasted_iota(jnp.int32, (1, _L), 1)
    cb = jnp.where((col1 >= start) & (col1 < end), 0.0, -1e30)

    s = jax.lax.dot_general(q, k, (((1,), (1,)), ((), ())),
                            preferred_element_type=jnp.float32)
    s = s + cb
    m = jnp.max(s, axis=1, keepdims=True)
    p = jnp.exp(s - m)
    sii = jnp.sum(q * k, axis=1, keepdims=True)
    pii = jnp.exp(sii - m)
    denom = jnp.sum(p, axis=1, keepdims=True) - pii
    o = (jnp.dot(p, v, preferred_element_type=jnp.float32) - pii * v) / denom

    row1 = sc + jax.lax.broadcasted_iota(jnp.int32, (_L, 1), 0)
    row_valid = (row1 >= start) & (row1 < end)
    cur = out_ref[pl.ds(sc, _L), :]
    out_ref[pl.ds(sc, _L), :] = jnp.where(row_valid, o, cur)


def kernel(embs_local_global, cu_seqlens, Wq, Wk, Wv, bq, bk, bv):
    t, d = embs_local_global.shape
    nseg = cu_seqlens.shape[0] - 1
    w = jnp.concatenate([Wq, Wk, Wv], axis=1)
    bias = jnp.concatenate([bq, bk, bv]).reshape(1, 3 * d)
    full = lambda shape: pl.BlockSpec(shape, lambda b: (0,) * len(shape))
    return pl.pallas_call(
        _attn_kernel,
        grid=(nseg,),
        in_specs=[
            pl.BlockSpec(memory_space=pltpu.SMEM),
            full((t, d)),
            full((d, 3 * d)),
            full((1, 3 * d)),
        ],
        out_specs=full((t, d)),
        out_shape=jax.ShapeDtypeStruct((t, d), jnp.float32),
        compiler_params=pltpu.CompilerParams(
            dimension_semantics=("arbitrary",)),
    )(cu_seqlens, embs_local_global, w, bias)


# speedup vs baseline: 1.1650x; 1.0943x over previous
import jax
import jax.numpy as jnp
from jax.experimental import pallas as pl
from jax.experimental.pallas import tpu as pltpu

_L = 512


def _attn_kernel(cu_ref, embs_ref, wq_ref, wk_ref, wv_ref, bq_ref, bk_ref,
                 bv_ref, out_ref):
    b = pl.program_id(0)
    t = embs_ref.shape[0]
    start = cu_ref[b]
    end = cu_ref[b + 1]
    sc = jnp.minimum(start, t - _L)

    x = embs_ref[pl.ds(sc, _L), :]
    q = jnp.dot(x, wq_ref[...], preferred_element_type=jnp.float32) + bq_ref[...]
    k = jnp.dot(x, wk_ref[...], preferred_element_type=jnp.float32) + bk_ref[...]
    v = jnp.dot(x, wv_ref[...], preferred_element_type=jnp.float32) + bv_ref[...]

    row_g = sc + jax.lax.broadcasted_iota(jnp.int32, (_L, _L), 0)
    col_g = sc + jax.lax.broadcasted_iota(jnp.int32, (_L, _L), 1)

    s = jax.lax.dot_general(q, k, (((1,), (1,)), ((), ())),
                            preferred_element_type=jnp.float32)
    mask = (col_g >= start) & (col_g < end) & (col_g != row_g)
    s = jnp.where(mask, s, -1e30)
    m = jnp.max(s, axis=1, keepdims=True)
    p = jnp.exp(s - m)
    attn = p / jnp.sum(p, axis=1, keepdims=True)
    o = jnp.dot(attn, v, preferred_element_type=jnp.float32)

    row1 = sc + jax.lax.broadcasted_iota(jnp.int32, (_L, 1), 0)
    row_valid = (row1 >= start) & (row1 < end)
    cur = out_ref[pl.ds(sc, _L), :]
    out_ref[pl.ds(sc, _L), :] = jnp.where(row_valid, o, cur)


def kernel(embs_local_global, cu_seqlens, Wq, Wk, Wv, bq, bk, bv):
    t, d = embs_local_global.shape
    nseg = cu_seqlens.shape[0] - 1
    bq2 = bq.reshape(1, d)
    bk2 = bk.reshape(1, d)
    bv2 = bv.reshape(1, d)
    full = lambda shape: pl.BlockSpec(shape, lambda b: (0,) * len(shape))
    return pl.pallas_call(
        _attn_kernel,
        grid=(nseg,),
        in_specs=[
            pl.BlockSpec(memory_space=pltpu.SMEM),
            full((t, d)),
            full((d, d)),
            full((d, d)),
            full((d, d)),
            full((1, d)),
            full((1, d)),
            full((1, d)),
        ],
        out_specs=full((t, d)),
        out_shape=jax.ShapeDtypeStruct((t, d), jnp.float32),
        compiler_params=pltpu.CompilerParams(
            dimension_semantics=("arbitrary",)),
    )(cu_seqlens, embs_local_global, Wq, Wk, Wv, bq2, bk2, bv2)


# speedup vs baseline: 1.4866x; 1.2761x over previous
import jax
import jax.numpy as jnp
from jax.experimental import pallas as pl
from jax.experimental.pallas import tpu as pltpu

_L = 512


def _attn_kernel(cu_ref, embs_ref, wq_ref, wk_ref, wv_ref, bq_ref, bk_ref,
                 bv_ref, out_ref):
    b = pl.program_id(0)
    t = embs_ref.shape[0]
    start = cu_ref[b]
    end = cu_ref[b + 1]
    length = end - start

    def tier_body(tier, ws):
        def body():
            x = embs_ref[pl.ds(ws, tier), :]
            q = jnp.dot(x, wq_ref[...],
                        preferred_element_type=jnp.float32) + bq_ref[...]
            k = jnp.dot(x, wk_ref[...],
                        preferred_element_type=jnp.float32) + bk_ref[...]
            v = jnp.dot(x, wv_ref[...],
                        preferred_element_type=jnp.float32) + bv_ref[...]

            row_g = ws + jax.lax.broadcasted_iota(jnp.int32, (tier, tier), 0)
            col_g = ws + jax.lax.broadcasted_iota(jnp.int32, (tier, tier), 1)

            s = jax.lax.dot_general(q, k, (((1,), (1,)), ((), ())),
                                    preferred_element_type=jnp.float32)
            mask = (col_g >= start) & (col_g < end) & (col_g != row_g)
            s = jnp.where(mask, s, -1e30)
            m = jnp.max(s, axis=1, keepdims=True)
            p = jnp.exp(s - m)
            attn = p / jnp.sum(p, axis=1, keepdims=True)
            o = jnp.dot(attn, v, preferred_element_type=jnp.float32)

            row1 = ws + jax.lax.broadcasted_iota(jnp.int32, (tier, 1), 0)
            row_valid = (row1 >= start) & (row1 < end)
            cur = out_ref[pl.ds(ws, tier), :]
            out_ref[pl.ds(ws, tier), :] = jnp.where(row_valid, o, cur)

        return body

    cond_a = (length <= 256) & (start <= t - 256)
    cond_b = jnp.logical_not(cond_a) & (length <= 384) & (start <= t - 384)
    cond_c = jnp.logical_not(cond_a | cond_b)
    pl.when(cond_a)(tier_body(256, start))
    pl.when(cond_b)(tier_body(384, start))
    pl.when(cond_c)(tier_body(_L, jnp.minimum(start, t - _L)))


def kernel(embs_local_global, cu_seqlens, Wq, Wk, Wv, bq, bk, bv):
    t, d = embs_local_global.shape
    nseg = cu_seqlens.shape[0] - 1
    bq2 = bq.reshape(1, d)
    bk2 = bk.reshape(1, d)
    bv2 = bv.reshape(1, d)
    full = lambda shape: pl.BlockSpec(shape, lambda b: (0,) * len(shape))
    return pl.pallas_call(
        _attn_kernel,
        grid=(nseg,),
        in_specs=[
            pl.BlockSpec(memory_space=pltpu.SMEM),
            full((t, d)),
            full((d, d)),
            full((d, d)),
            full((d, d)),
            full((1, d)),
            full((1, d)),
            full((1, d)),
        ],
        out_specs=full((t, d)),
        out_shape=jax.ShapeDtypeStruct((t, d), jnp.float32),
        compiler_params=pltpu.CompilerParams(
            dimension_semantics=("arbitrary",)),
    )(cu_seqlens, embs_local_global, Wq, Wk, Wv, bq2, bk2, bv2)


# speedup vs baseline: 1.5958x; 1.0734x over previous
import jax
import jax.numpy as jnp
from jax.experimental import pallas as pl
from jax.experimental.pallas import tpu as pltpu

_L = 512


def _attn_kernel(cu_ref, embs_ref, wq_ref, wk_ref, wv_ref, bq_ref, bk_ref,
                 bv_ref, out_ref):
    b = pl.program_id(0)
    t = embs_ref.shape[0]
    start = cu_ref[b]
    end = cu_ref[b + 1]
    length = end - start

    def tier_body(tier, ws):
        def body():
            x = embs_ref[pl.ds(ws, tier), :]
            q = jnp.dot(x, wq_ref[...],
                        preferred_element_type=jnp.float32) + bq_ref[...]
            k = jnp.dot(x, wk_ref[...], preferred_element_type=jnp.float32)
            v = jnp.dot(x, wv_ref[...],
                        preferred_element_type=jnp.float32) + bv_ref[...]

            ri = jax.lax.broadcasted_iota(jnp.int32, (tier, 1), 0)
            ci = jax.lax.broadcasted_iota(jnp.int32, (1, tier), 1)
            colv = ((ws + ci) >= start) & ((ws + ci) < end)
            cb = jnp.where(colv, 0.0, -1e30)

            s = jax.lax.dot_general(q, k, (((1,), (1,)), ((), ())),
                                    preferred_element_type=jnp.float32)
            s = jnp.where(ri != ci, s + cb, -1e30)
            m = jnp.max(s, axis=1, keepdims=True)
            p = jnp.exp(s - m)
            denom = jnp.sum(p, axis=1, keepdims=True)
            o = jnp.dot(p, v, preferred_element_type=jnp.float32) / denom

            row1 = ws + ri
            row_valid = (row1 >= start) & (row1 < end)
            cur = out_ref[pl.ds(ws, tier), :]
            out_ref[pl.ds(ws, tier), :] = jnp.where(row_valid, o, cur)

        return body

    cond_a = (length <= 256) & (start <= t - 256)
    cond_b = jnp.logical_not(cond_a) & (length <= 384) & (start <= t - 384)
    cond_c = jnp.logical_not(cond_a | cond_b)
    pl.when(cond_a)(tier_body(256, start))
    pl.when(cond_b)(tier_body(384, start))
    pl.when(cond_c)(tier_body(_L, jnp.minimum(start, t - _L)))


def kernel(embs_local_global, cu_seqlens, Wq, Wk, Wv, bq, bk, bv):
    t, d = embs_local_global.shape
    nseg = cu_seqlens.shape[0] - 1
    bq2 = bq.reshape(1, d)
    bk2 = bk.reshape(1, d)
    bv2 = bv.reshape(1, d)
    full = lambda shape: pl.BlockSpec(shape, lambda b: (0,) * len(shape))
    return pl.pallas_call(
        _attn_kernel,
        grid=(nseg,),
        in_specs=[
            pl.BlockSpec(memory_space=pltpu.SMEM),
            full((t, d)),
            full((d, d)),
            full((d, d)),
            full((d, d)),
            full((1, d)),
            full((1, d)),
            full((1, d)),
        ],
        out_specs=full((t, d)),
        out_shape=jax.ShapeDtypeStruct((t, d), jnp.float32),
        compiler_params=pltpu.CompilerParams(
            dimension_semantics=("arbitrary",)),
    )(cu_seqlens, embs_local_global, Wq, Wk, Wv, bq2, bk2, bv2)
